# baseline (speedup 1.0000x reference)
import jax
import jax.numpy as jnp
from jax.experimental import pallas as pl
from jax.experimental.pallas import tpu as pltpu

B, S, DIM, E, K = 2, 32, 512, 64, 2
H = 4 * DIM
T = B * S

_SQRT_HALF = 0.7071067811865476


def _gelu(t):
    return 0.5 * t * (1.0 + jax.lax.erf(t * _SQRT_HALF))


def _moe_kernel(x_ref, gw1_ref, gb1_ref, gw2_ref, gb2_ref, gw3_ref, gb3_ref,
                eb1_ref, eb2_ref, ew1_hbm, ew2_hbm, out_ref):
    hi = None
    xt = x_ref[...]

    g = _gelu(jnp.dot(xt, gw1_ref[...], precision=hi,
                      preferred_element_type=jnp.float32) + gb1_ref[0])
    g = _gelu(jnp.dot(g, gw2_ref[...], precision=hi,
                      preferred_element_type=jnp.float32) + gb2_ref[0])
    logits = jax.nn.sigmoid(jnp.dot(g, gw3_ref[...], precision=hi,
                                    preferred_element_type=jnp.float32)
                            + gb3_ref[0])
    iota = jax.lax.broadcasted_iota(jnp.int32, (T, E), 1)
    v1 = jnp.max(logits, axis=1, keepdims=True)
    i1 = jnp.min(jnp.where(logits == v1, iota, E), axis=1, keepdims=True)
    masked = jnp.where(iota == i1, -jnp.inf, logits)
    v2 = jnp.max(masked, axis=1, keepdims=True)
    i2 = jnp.min(jnp.where(masked == v2, iota, E), axis=1, keepdims=True)
    s = v1 + v2
    v1n = v1 / s
    v2n = v2 / s

    out_ref[...] = jnp.zeros((T, DIM), jnp.float32)

    def expert_body(w1_ref, w2_ref):
        e = pl.program_id(0)
        w1 = w1_ref[0]
        w2 = w2_ref[0]
        h = _gelu(jax.lax.dot_general(xt, w1, (((1,), (1,)), ((), ())),
                                      precision=hi,
                                      preferred_element_type=jnp.float32)
                  + eb1_ref[e])
        o = _gelu(jnp.dot(h, w2, precision=hi,
                          preferred_element_type=jnp.float32) + eb2_ref[e])
        scale = (jnp.where(i1 == e, v1n, 0.0)
                 + jnp.where(i2 == e, v2n, 0.0))
        out_ref[...] += scale * o

    pltpu.emit_pipeline(
        expert_body,
        grid=(E,),
        in_specs=[
            pl.BlockSpec((1, H, DIM), lambda e: (e, 0, 0),
                         pipeline_mode=pl.Buffered(buffer_count=4)),
            pl.BlockSpec((1, H, DIM), lambda e: (e, 0, 0),
                         pipeline_mode=pl.Buffered(buffer_count=4)),
        ],
    )(ew1_hbm, ew2_hbm)


def kernel(x, gw1, gb1, gw2, gb2, gw3, gb3, ew1, ew2, eb1, eb2):
    xt = x.reshape(T, DIM)
    eb1r = eb1.reshape(E, 1, H)
    eb2r = eb2.reshape(E, 1, DIM)

    out = pl.pallas_call(
        _moe_kernel,
        in_specs=[
            pl.BlockSpec((T, DIM), lambda: (0, 0)),
            pl.BlockSpec((DIM, H), lambda: (0, 0)),
            pl.BlockSpec((1, H), lambda: (0, 0)),
            pl.BlockSpec((H, H), lambda: (0, 0)),
            pl.BlockSpec((1, H), lambda: (0, 0)),
            pl.BlockSpec((H, E), lambda: (0, 0)),
            pl.BlockSpec((1, E), lambda: (0, 0)),
            pl.BlockSpec((E, 1, H), lambda: (0, 0, 0)),
            pl.BlockSpec((E, 1, DIM), lambda: (0, 0, 0)),
            pl.BlockSpec(memory_space=pl.ANY),
            pl.BlockSpec(memory_space=pl.ANY),
        ],
        out_specs=pl.BlockSpec((T, DIM), lambda: (0, 0)),
        out_shape=jax.ShapeDtypeStruct((T, DIM), jnp.float32),
    )(xt, gw1, gb1.reshape(1, H), gw2, gb2.reshape(1, H), gw3,
      gb3.reshape(1, E), eb1r, eb2r, ew1, ew2)

    return out.reshape(B, S, DIM)


# speedup vs baseline: 1.0319x; 1.0319x over previous
import jax
import jax.numpy as jnp
from jax.experimental import pallas as pl
from jax.experimental.pallas import tpu as pltpu

B, S, DIM, E, K = 2, 32, 512, 64, 2
H = 4 * DIM
T = B * S

_SQRT_HALF = 0.7071067811865476


def _gelu(t):
    return 0.5 * t * (1.0 + jax.lax.erf(t * _SQRT_HALF))


def _moe_kernel(x_ref, gw1_ref, gb1_ref, gw2_ref, gb2_ref, gw3_ref, gb3_ref,
                eb1_ref, eb2_ref, ew1_hbm, ew2_hbm, out_ref):
    hi = None
    xt = x_ref[...]

    g = _gelu(jnp.dot(xt, gw1_ref[...], precision=hi,
                      preferred_element_type=jnp.float32) + gb1_ref[0])
    g = _gelu(jnp.dot(g, gw2_ref[...], precision=hi,
                      preferred_element_type=jnp.float32) + gb2_ref[0])
    logits = jax.nn.sigmoid(jnp.dot(g, gw3_ref[...], precision=hi,
                                    preferred_element_type=jnp.float32)
                            + gb3_ref[0])
    iota = jax.lax.broadcasted_iota(jnp.int32, (T, E), 1)
    v1 = jnp.max(logits, axis=1, keepdims=True)
    i1 = jnp.min(jnp.where(logits == v1, iota, E), axis=1, keepdims=True)
    masked = jnp.where(iota == i1, -jnp.inf, logits)
    v2 = jnp.max(masked, axis=1, keepdims=True)
    i2 = jnp.min(jnp.where(masked == v2, iota, E), axis=1, keepdims=True)
    s = v1 + v2
    v1n = v1 / s
    v2n = v2 / s

    out_ref[...] = jnp.zeros((T, DIM), jnp.float32)

    def expert_body(w1_ref, w2_ref):
        e = pl.program_id(0)
        w1 = w1_ref[0]
        w2 = w2_ref[0]
        h = _gelu(jax.lax.dot_general(xt, w1, (((1,), (1,)), ((), ())),
                                      precision=hi,
                                      preferred_element_type=jnp.float32)
                  + eb1_ref[e])
        o = _gelu(jnp.dot(h, w2, precision=hi,
                          preferred_element_type=jnp.float32) + eb2_ref[e])
        scale = (jnp.where(i1 == e, v1n, 0.0)
                 + jnp.where(i2 == e, v2n, 0.0))
        out_ref[...] += scale * o

    pltpu.emit_pipeline(
        expert_body,
        grid=(E,),
        in_specs=[
            pl.BlockSpec((1, H, DIM), lambda e: (e, 0, 0),
                         pipeline_mode=pl.Buffered(buffer_count=2)),
            pl.BlockSpec((1, H, DIM), lambda e: (e, 0, 0),
                         pipeline_mode=pl.Buffered(buffer_count=2)),
        ],
    )(ew1_hbm, ew2_hbm)


def kernel(x, gw1, gb1, gw2, gb2, gw3, gb3, ew1, ew2, eb1, eb2):
    xt = x.reshape(T, DIM)
    eb1r = eb1.reshape(E, 1, H)
    eb2r = eb2.reshape(E, 1, DIM)

    out = pl.pallas_call(
        _moe_kernel,
        in_specs=[
            pl.BlockSpec((T, DIM), lambda: (0, 0)),
            pl.BlockSpec((DIM, H), lambda: (0, 0)),
            pl.BlockSpec((1, H), lambda: (0, 0)),
            pl.BlockSpec((H, H), lambda: (0, 0)),
            pl.BlockSpec((1, H), lambda: (0, 0)),
            pl.BlockSpec((H, E), lambda: (0, 0)),
            pl.BlockSpec((1, E), lambda: (0, 0)),
            pl.BlockSpec((E, 1, H), lambda: (0, 0, 0)),
            pl.BlockSpec((E, 1, DIM), lambda: (0, 0, 0)),
            pl.BlockSpec(memory_space=pl.ANY),
            pl.BlockSpec(memory_space=pl.ANY),
        ],
        out_specs=pl.BlockSpec((T, DIM), lambda: (0, 0)),
        out_shape=jax.ShapeDtypeStruct((T, DIM), jnp.float32),
    )(xt, gw1, gb1.reshape(1, H), gw2, gb2.reshape(1, H), gw3,
      gb3.reshape(1, E), eb1r, eb2r, ew1, ew2)

    return out.reshape(B, S, DIM)


# speedup vs baseline: 1.0561x; 1.0234x over previous
import jax
import jax.numpy as jnp
from jax.experimental import pallas as pl
from jax.experimental.pallas import tpu as pltpu

B, S, DIM, E, K = 2, 32, 512, 64, 2
H = 4 * DIM
T = B * S
H2 = H // 2

_SQRT_HALF = 0.7071067811865476


def _gelu(t):
    return 0.5 * t * (1.0 + jax.lax.erf(t * _SQRT_HALF))


def _moe_kernel(x_ref, gw1_ref, gb1_ref, gw2_ref, gb2_ref, gw3_ref, gb3_ref,
                eb1_ref, eb2_ref, ew1_hbm, ew2_hbm, out_ref):
    hi = None
    xt = x_ref[...]

    g = _gelu(jnp.dot(xt, gw1_ref[...], precision=hi,
                      preferred_element_type=jnp.float32) + gb1_ref[0])
    g = _gelu(jnp.dot(g, gw2_ref[...], precision=hi,
                      preferred_element_type=jnp.float32) + gb2_ref[0])
    logits = jax.nn.sigmoid(jnp.dot(g, gw3_ref[...], precision=hi,
                                    preferred_element_type=jnp.float32)
                            + gb3_ref[0])
    iota = jax.lax.broadcasted_iota(jnp.int32, (T, E), 1)
    v1 = jnp.max(logits, axis=1, keepdims=True)
    i1 = jnp.min(jnp.where(logits == v1, iota, E), axis=1, keepdims=True)
    masked = jnp.where(iota == i1, -jnp.inf, logits)
    v2 = jnp.max(masked, axis=1, keepdims=True)
    i2 = jnp.min(jnp.where(masked == v2, iota, E), axis=1, keepdims=True)
    s = v1 + v2
    v1n = v1 / s
    v2n = v2 / s

    out_ref[...] = jnp.zeros((T, DIM), jnp.float32)

    def expert_body(w1a_ref, w1b_ref, w2a_ref, w2b_ref):
        e = pl.program_id(0)
        b1 = eb1_ref[pl.ds(e, 1), 0, :]
        ha = _gelu(jax.lax.dot_general(xt, w1a_ref[0, 0],
                                       (((1,), (1,)), ((), ())),
                                       precision=hi,
                                       preferred_element_type=jnp.float32)
                   + b1[:, :H2])
        hb = _gelu(jax.lax.dot_general(xt, w1b_ref[0, 0],
                                       (((1,), (1,)), ((), ())),
                                       precision=hi,
                                       preferred_element_type=jnp.float32)
                   + b1[:, H2:])
        opre = (jnp.dot(ha, w2a_ref[0, 0], precision=hi,
                        preferred_element_type=jnp.float32)
                + jnp.dot(hb, w2b_ref[0, 0], precision=hi,
                          preferred_element_type=jnp.float32))
        o = _gelu(opre + eb2_ref[pl.ds(e, 1), 0, :])
        scale = (jnp.where(i1 == e, v1n, 0.0)
                 + jnp.where(i2 == e, v2n, 0.0))
        out_ref[...] += scale * o

    wspec = lambda q: pl.BlockSpec((1, 1, H2, DIM),
                                   lambda e, _q=q: (e, _q, 0, 0),
                                   pipeline_mode=pl.Buffered(buffer_count=3))
    pltpu.emit_pipeline(
        expert_body,
        grid=(E,),
        in_specs=[wspec(0), wspec(1), wspec(0), wspec(1)],
    )(ew1_hbm, ew1_hbm, ew2_hbm, ew2_hbm)


def kernel(x, gw1, gb1, gw2, gb2, gw3, gb3, ew1, ew2, eb1, eb2):
    xt = x.reshape(T, DIM)
    eb1r = eb1.reshape(E, 1, H)
    eb2r = eb2.reshape(E, 1, DIM)
    ew1r = ew1.reshape(E, 2, H2, DIM)
    ew2r = ew2.reshape(E, 2, H2, DIM)

    out = pl.pallas_call(
        _moe_kernel,
        in_specs=[
            pl.BlockSpec((T, DIM), lambda: (0, 0)),
            pl.BlockSpec((DIM, H), lambda: (0, 0)),
            pl.BlockSpec((1, H), lambda: (0, 0)),
            pl.BlockSpec((H, H), lambda: (0, 0)),
            pl.BlockSpec((1, H), lambda: (0, 0)),
            pl.BlockSpec((H, E), lambda: (0, 0)),
            pl.BlockSpec((1, E), lambda: (0, 0)),
            pl.BlockSpec((E, 1, H), lambda: (0, 0, 0)),
            pl.BlockSpec((E, 1, DIM), lambda: (0, 0, 0)),
            pl.BlockSpec(memory_space=pl.ANY),
            pl.BlockSpec(memory_space=pl.ANY),
        ],
        out_specs=pl.BlockSpec((T, DIM), lambda: (0, 0)),
        out_shape=jax.ShapeDtypeStruct((T, DIM), jnp.float32),
    )(xt, gw1, gb1.reshape(1, H), gw2, gb2.reshape(1, H), gw3,
      gb3.reshape(1, E), eb1r, eb2r, ew1r, ew2r)

    return out.reshape(B, S, DIM)
